# Initial kernel scaffold; baseline (speedup 1.0000x reference)
import jax
import jax.numpy as jnp
from jax.experimental import pallas as pl


def kernel(x, edge_index, W1a, b1a, W1b, b1b, W2a, b2a, W2b, b2b, Wl, bl, Wd1, bd1, Wd2, bd2):
    raise NotImplementedError("write your pallas kernel here")


# speedup vs baseline: 1.1280x; 1.1280x over previous
import functools

import jax
import jax.numpy as jnp
from jax import lax
from jax.experimental import pallas as pl
from jax.experimental.pallas import tpu as pltpu
from jax.experimental.pallas import tpu_sc as plsc

NP = 10240
NC = 2
NS = 16
EB = 128
CW = 128


def _mm(lhs, rhs, *, bpre=None, bpost=None, act=None, out_chunked=False,
        bm=512, bn=512, bk=512):
    chunked_lhs = lhs.ndim == 3
    if chunked_lhs:
        K = lhs.shape[0] * CW
        M = lhs.shape[1]
    else:
        M, K = lhs.shape
    Nn = rhs.shape[1]
    nm, nn, nk = M // bm, Nn // bn, K // bk
    C = bk // CW
    CO = bn // CW

    in_specs = []
    args = []
    if chunked_lhs:
        in_specs.append(pl.BlockSpec((C, bm, CW), lambda m, n, k: (k, m, 0)))
    else:
        in_specs.append(pl.BlockSpec((bm, bk), lambda m, n, k: (m, k)))
    args.append(lhs)
    in_specs.append(pl.BlockSpec((bk, bn), lambda m, n, k: (k, n)))
    args.append(rhs)
    if bpre is not None:
        in_specs.append(pl.BlockSpec((1, bk), lambda m, n, k: (0, k)))
        args.append(bpre.reshape(1, K))
    if bpost is not None:
        in_specs.append(pl.BlockSpec((1, bn), lambda m, n, k: (0, n)))
        args.append(bpost.reshape(1, Nn))

    if out_chunked:
        out_spec = pl.BlockSpec((CO, bm, CW), lambda m, n, k: (n, m, 0))
        out_shape = jax.ShapeDtypeStruct((Nn // CW, M, CW), jnp.float32)
    else:
        out_spec = pl.BlockSpec((bm, bn), lambda m, n, k: (m, n))
        out_shape = jax.ShapeDtypeStruct((M, Nn), jnp.float32)

    def body(*refs):
        it = iter(refs)
        lhs_ref = next(it)
        rhs_ref = next(it)
        bpre_ref = next(it) if bpre is not None else None
        bpost_ref = next(it) if bpost is not None else None
        out_ref = next(it)
        acc_ref = next(it)

        k = pl.program_id(2)

        @pl.when(k == 0)
        def _():
            acc_ref[...] = jnp.zeros_like(acc_ref)

        if chunked_lhs:
            a = jnp.concatenate([lhs_ref[c] for c in range(C)], axis=1)
            a = jnp.maximum(a + bpre_ref[...], 0.0)
        else:
            a = lhs_ref[...]
        acc_ref[...] += jnp.dot(a, rhs_ref[...],
                                preferred_element_type=jnp.float32)

        @pl.when(k == nk - 1)
        def _():
            z = acc_ref[...]
            if bpost is not None:
                z = z + bpost_ref[...]
            if act == "relu":
                z = jnp.maximum(z, 0.0)
            elif act == "leaky":
                z = jnp.where(z >= 0.0, z, 0.01 * z)
            if out_chunked:
                for co in range(CO):
                    out_ref[co, :, :] = z[:, co * CW:(co + 1) * CW]
            else:
                out_ref[...] = z

    return pl.pallas_call(
        body,
        grid=(nm, nn, nk),
        in_specs=in_specs,
        out_specs=out_spec,
        out_shape=out_shape,
        scratch_shapes=[pltpu.VMEM((bm, bn), jnp.float32)],
        compiler_params=pltpu.CompilerParams(
            dimension_semantics=("parallel", "parallel", "arbitrary")),
    )(*args)


def _segsum_sc(yt, srcs, dsts, nch):
    NB = srcs.shape[1]
    cpc = nch // NC
    rows = NP // NS

    mesh = plsc.VectorSubcoreMesh(core_axis_name="c", subcore_axis_name="s",
                                  num_cores=NC, num_subcores=NS)

    @functools.partial(
        pl.kernel,
        out_type=jax.ShapeDtypeStruct((nch * NP, CW), jnp.float32),
        mesh=mesh,
        scratch_types=[
            pltpu.VMEM_SHARED((NP, CW), jnp.float32),
            pltpu.VMEM((NB, EB), jnp.int32),
            pltpu.VMEM((NB, EB), jnp.int32),
            pltpu.VMEM((NB, EB), jnp.int32),
            pltpu.VMEM((2, EB, CW), jnp.float32),
            pltpu.SemaphoreType.DMA,
        ],
    )
    def seg_kernel(yt_h, src_h, dst_h, out_h, acc, srcv, dstv, sabs, buf, sem):
        c = lax.axis_index("c")
        s = lax.axis_index("s")
        pltpu.sync_copy(src_h.at[s], srcv)
        pltpu.sync_copy(dst_h.at[s], dstv)
        for cl in range(cpc):
            ci = c * cpc + cl
            base = ci * NP
            for j in range(NB):
                for q in range(EB // 16):
                    sabs[j, pl.ds(q * 16, 16)] = (
                        srcv[j, pl.ds(q * 16, 16)] + base)
            pltpu.sync_copy(yt_h.at[pl.ds(base + s * rows, rows)],
                            acc.at[pl.ds(s * rows, rows)])
            plsc.subcore_barrier()
            for b in range(NB):
                pltpu.async_copy(yt_h.at[sabs.at[b]], buf.at[b % 2],
                                 sem).wait()
                pltpu.sync_copy(buf.at[b % 2], acc.at[dstv.at[b]], add=True)
            plsc.subcore_barrier()
            pltpu.sync_copy(acc.at[pl.ds(s * rows, rows)],
                            out_h.at[pl.ds(base + s * rows, rows)])

    return seg_kernel(yt, srcs, dsts)


def _prep_edges(edge_index, n):
    src = edge_index[0].astype(jnp.int32)
    dst = edge_index[1].astype(jnp.int32)
    e = src.shape[0]
    ep = ((e + NS * EB - 1) // (NS * EB)) * (NS * EB)
    pad = ep - e
    pad_idx = n + (jnp.arange(pad, dtype=jnp.int32) % (NP - n))
    srcp = jnp.concatenate([src, pad_idx]).reshape(NS, ep // (NS * EB), EB)
    dstp = jnp.concatenate([dst, pad_idx]).reshape(NS, ep // (NS * EB), EB)
    return srcp, dstp


def kernel(x, edge_index, W1a, b1a, W1b, b1b, W2a, b2a, W2b, b2b,
           Wl, bl, Wd1, bd1, Wd2, bd2):
    n = x.shape[0]
    x_p = jnp.pad(x, ((0, NP - n), (0, 0)))
    srcs, dsts = _prep_edges(edge_index, n)

    y1 = _mm(x_p, W1a, out_chunked=True)
    h1pre = _segsum_sc(y1.reshape(-1, CW), srcs, dsts, W1a.shape[1] // CW)
    h1pre = h1pre.reshape(W1a.shape[1] // CW, NP, CW)
    h1 = _mm(h1pre, W1b, bpre=b1a, bpost=b1b, act="relu")

    y2 = _mm(h1, W2a, out_chunked=True)
    h2pre = _segsum_sc(y2.reshape(-1, CW), srcs, dsts, W2a.shape[1] // CW)
    h2pre = h2pre.reshape(W2a.shape[1] // CW, NP, CW)
    h2 = _mm(h2pre, W2b, bpre=b2a, bpost=b2b)

    enc = _mm(h2, Wl, bpost=bl)
    d = _mm(enc, Wd1, bpost=bd1, act="leaky")
    dec = _mm(d, Wd2, bpost=bd2)

    return (dec[:n], enc[:n])


# speedup vs baseline: 2.4616x; 2.1823x over previous
import functools

import jax
import jax.numpy as jnp
from jax import lax
from jax.experimental import pallas as pl
from jax.experimental.pallas import tpu as pltpu
from jax.experimental.pallas import tpu_sc as plsc

NP = 10240
NC = 2
NS = 16
EB = 128
CW = 128


def _mm(lhs, rhs, *, bpre=None, bpost=None, act=None, out_chunked=False,
        out_dtype=jnp.float32, bm=1024, bn=1024):
    chunked_lhs = lhs.ndim == 3
    if chunked_lhs:
        K = lhs.shape[0] * CW
        M = lhs.shape[1]
    else:
        M, K = lhs.shape
    Nn = rhs.shape[1]
    bn = min(bn, Nn)
    nm, nn = M // bm, Nn // bn
    C = K // CW
    CO = bn // CW

    in_specs = []
    args = []
    if chunked_lhs:
        in_specs.append(pl.BlockSpec((C, bm, CW), lambda m, n: (0, m, 0)))
    else:
        in_specs.append(pl.BlockSpec((bm, K), lambda m, n: (m, 0)))
    args.append(lhs)
    in_specs.append(pl.BlockSpec((K, bn), lambda m, n: (0, n)))
    args.append(rhs)
    if bpre is not None:
        in_specs.append(pl.BlockSpec((1, K), lambda m, n: (0, 0)))
        args.append(bpre.reshape(1, K))
    if bpost is not None:
        in_specs.append(pl.BlockSpec((1, bn), lambda m, n: (0, n)))
        args.append(bpost.reshape(1, Nn))

    if out_chunked:
        out_spec = pl.BlockSpec((CO, bm, CW), lambda m, n: (n, m, 0))
        out_shape = jax.ShapeDtypeStruct((Nn // CW, M, CW), jnp.float32)
    else:
        out_spec = pl.BlockSpec((bm, bn), lambda m, n: (m, n))
        out_shape = jax.ShapeDtypeStruct((M, Nn), out_dtype)

    def body(*refs):
        it = iter(refs)
        lhs_ref = next(it)
        rhs_ref = next(it)
        bpre_ref = next(it) if bpre is not None else None
        bpost_ref = next(it) if bpost is not None else None
        out_ref = next(it)

        if chunked_lhs:
            a = jnp.concatenate([lhs_ref[c] for c in range(C)], axis=1)
            a = jnp.maximum(a + bpre_ref[...], 0.0)
        else:
            a = lhs_ref[...]
        a = a.astype(jnp.bfloat16)
        z = jnp.dot(a, rhs_ref[...], preferred_element_type=jnp.float32)
        if bpost is not None:
            z = z + bpost_ref[...]
        if act == "relu":
            z = jnp.maximum(z, 0.0)
        elif act == "leaky":
            z = jnp.where(z >= 0.0, z, 0.01 * z)
        if out_chunked:
            for co in range(CO):
                out_ref[co, :, :] = z[:, co * CW:(co + 1) * CW]
        else:
            out_ref[...] = z.astype(out_dtype)

    return pl.pallas_call(
        body,
        grid=(nm, nn),
        in_specs=in_specs,
        out_specs=out_spec,
        out_shape=out_shape,
        compiler_params=pltpu.CompilerParams(
            dimension_semantics=("parallel", "parallel")),
    )(*args)


def _segsum_sc(yt, srcs, dsts, nch):
    NB = srcs.shape[1]
    cpc = nch // NC
    rows = NP // NS

    mesh = plsc.VectorSubcoreMesh(core_axis_name="c", subcore_axis_name="s",
                                  num_cores=NC, num_subcores=NS)

    @functools.partial(
        pl.kernel,
        out_type=jax.ShapeDtypeStruct((nch * NP, CW), jnp.float32),
        mesh=mesh,
        scratch_types=[
            pltpu.VMEM_SHARED((NP, CW), jnp.float32),
            pltpu.VMEM((NB, EB), jnp.int32),
            pltpu.VMEM((NB, EB), jnp.int32),
            pltpu.VMEM((NB, EB), jnp.int32),
            pltpu.VMEM((2, EB, CW), jnp.float32),
            pltpu.SemaphoreType.DMA,
        ],
    )
    def seg_kernel(yt_h, src_h, dst_h, out_h, acc, srcv, dstv, sabs, buf, sem):
        c = lax.axis_index("c")
        s = lax.axis_index("s")
        pltpu.sync_copy(src_h.at[s], srcv)
        pltpu.sync_copy(dst_h.at[s], dstv)
        for cl in range(cpc):
            ci = c * cpc + cl
            base = ci * NP
            for j in range(NB):
                for q in range(EB // 16):
                    sabs[j, pl.ds(q * 16, 16)] = (
                        srcv[j, pl.ds(q * 16, 16)] + base)
            pltpu.sync_copy(yt_h.at[pl.ds(base + s * rows, rows)],
                            acc.at[pl.ds(s * rows, rows)])
            plsc.subcore_barrier()
            for b in range(NB):
                pltpu.async_copy(yt_h.at[sabs.at[b]], buf.at[b % 2],
                                 sem).wait()
                pltpu.sync_copy(buf.at[b % 2], acc.at[dstv.at[b]], add=True)
            plsc.subcore_barrier()
            pltpu.sync_copy(acc.at[pl.ds(s * rows, rows)],
                            out_h.at[pl.ds(base + s * rows, rows)])

    return seg_kernel(yt, srcs, dsts)


def _prep_edges(edge_index, n):
    src = edge_index[0].astype(jnp.int32)
    dst = edge_index[1].astype(jnp.int32)
    e = src.shape[0]
    ep = ((e + NS * EB - 1) // (NS * EB)) * (NS * EB)
    pad = ep - e
    pad_idx = n + (jnp.arange(pad, dtype=jnp.int32) % (NP - n))
    srcp = jnp.concatenate([src, pad_idx]).reshape(NS, ep // (NS * EB), EB)
    dstp = jnp.concatenate([dst, pad_idx]).reshape(NS, ep // (NS * EB), EB)
    return srcp, dstp


def kernel(x, edge_index, W1a, b1a, W1b, b1b, W2a, b2a, W2b, b2b,
           Wl, bl, Wd1, bd1, Wd2, bd2):
    n = x.shape[0]
    x_p = jnp.pad(x, ((0, NP - n), (0, 0))).astype(jnp.bfloat16)
    srcs, dsts = _prep_edges(edge_index, n)
    W1a, W1b, W2a, W2b, Wl, Wd1, Wd2 = (
        w.astype(jnp.bfloat16) for w in (W1a, W1b, W2a, W2b, Wl, Wd1, Wd2))

    y1 = _mm(x_p, W1a, out_chunked=True)
    h1pre = _segsum_sc(y1.reshape(-1, CW), srcs, dsts, W1a.shape[1] // CW)
    h1pre = h1pre.reshape(W1a.shape[1] // CW, NP, CW)
    h1 = _mm(h1pre, W1b, bpre=b1a, bpost=b1b, act="relu",
             out_dtype=jnp.bfloat16)

    y2 = _mm(h1, W2a, out_chunked=True)
    h2pre = _segsum_sc(y2.reshape(-1, CW), srcs, dsts, W2a.shape[1] // CW)
    h2pre = h2pre.reshape(W2a.shape[1] // CW, NP, CW)
    h2 = _mm(h2pre, W2b, bpre=b2a, bpost=b2b,
             out_dtype=jnp.bfloat16)

    enc = _mm(h2, Wl, bpost=bl)
    d = _mm(enc, Wd1, bpost=bd1, act="leaky",
            out_dtype=jnp.bfloat16)
    dec = _mm(d, Wd2, bpost=bd2, bn=2048)

    return (dec[:n], enc[:n])


# speedup vs baseline: 2.5779x; 1.0473x over previous
import functools

import jax
import jax.numpy as jnp
from jax import lax
from jax.experimental import pallas as pl
from jax.experimental.pallas import tpu as pltpu
from jax.experimental.pallas import tpu_sc as plsc

NP = 10240
NC = 2
NS = 16
EB = 128
CW = 128


def _gin_mlp_mm(h_pre, Ws, bs, acts, out_chunked, out_dtype=jnp.float32,
                bm=512):
    C, M, _ = h_pre.shape
    K = C * CW
    nW = len(Ws)

    in_specs = [pl.BlockSpec((C, bm, CW), lambda m: (0, m, 0)),
                pl.BlockSpec((1, K), lambda m: (0, 0))]
    args = [h_pre, bs[0].reshape(1, K)]
    for W, b in zip(Ws, bs[1:]):
        kk, nn = W.shape
        in_specs.append(pl.BlockSpec((kk, nn), lambda m: (0, 0)))
        args.append(W)
        in_specs.append(pl.BlockSpec((1, nn), lambda m: (0, 0)))
        args.append(b.reshape(1, nn))

    out_specs = []
    out_shapes = []
    for i, oc in enumerate(out_chunked):
        nn = Ws[i].shape[1]
        if oc is None:
            continue
        if oc:
            out_specs.append(pl.BlockSpec((nn // CW, bm, CW),
                                          lambda m: (0, m, 0)))
            out_shapes.append(
                jax.ShapeDtypeStruct((nn // CW, M, CW), jnp.float32))
        else:
            out_specs.append(pl.BlockSpec((bm, nn), lambda m: (m, 0)))
            out_shapes.append(jax.ShapeDtypeStruct((M, nn), jnp.float32))

    def body(*refs):
        it = iter(refs)
        lhs_ref = next(it)
        bpre_ref = next(it)
        wrefs = []
        for _ in range(nW):
            wrefs.append((next(it), next(it)))
        orefs = [next(it) for _ in range(len(out_specs))]

        a = jnp.concatenate([lhs_ref[c] for c in range(C)], axis=1)
        a = jnp.maximum(a + bpre_ref[...], 0.0).astype(jnp.bfloat16)
        oi = 0
        for i, ((w_ref, b_ref), act) in enumerate(zip(wrefs, acts)):
            z = jnp.dot(a, w_ref[...], preferred_element_type=jnp.float32)
            z = z + b_ref[...]
            if act == "relu":
                z = jnp.maximum(z, 0.0)
            elif act == "leaky":
                z = jnp.where(z >= 0.0, z, 0.01 * z)
            oc = out_chunked[i]
            if oc is not None:
                if oc:
                    nn = z.shape[1]
                    for co in range(nn // CW):
                        orefs[oi][co, :, :] = z[:, co * CW:(co + 1) * CW]
                else:
                    orefs[oi][...] = z
                oi += 1
            a = z.astype(jnp.bfloat16)

    outs = pl.pallas_call(
        body,
        grid=(M // bm,),
        in_specs=in_specs,
        out_specs=out_specs,
        out_shape=out_shapes,
        compiler_params=pltpu.CompilerParams(
            dimension_semantics=("parallel",)),
    )(*args)
    return outs


def _mm(lhs, rhs, *, bpre=None, bpost=None, act=None, out_chunked=False,
        out_dtype=jnp.float32, bm=512, bn=2048):
    chunked_lhs = lhs.ndim == 3
    if chunked_lhs:
        K = lhs.shape[0] * CW
        M = lhs.shape[1]
    else:
        M, K = lhs.shape
    Nn = rhs.shape[1]
    bn = min(bn, Nn)
    nm, nn = M // bm, Nn // bn
    C = K // CW
    CO = bn // CW

    in_specs = []
    args = []
    if chunked_lhs:
        in_specs.append(pl.BlockSpec((C, bm, CW), lambda m, n: (0, m, 0)))
    else:
        in_specs.append(pl.BlockSpec((bm, K), lambda m, n: (m, 0)))
    args.append(lhs)
    in_specs.append(pl.BlockSpec((K, bn), lambda m, n: (0, n)))
    args.append(rhs)
    if bpre is not None:
        in_specs.append(pl.BlockSpec((1, K), lambda m, n: (0, 0)))
        args.append(bpre.reshape(1, K))
    if bpost is not None:
        in_specs.append(pl.BlockSpec((1, bn), lambda m, n: (0, n)))
        args.append(bpost.reshape(1, Nn))

    if out_chunked:
        out_spec = pl.BlockSpec((CO, bm, CW), lambda m, n: (n, m, 0))
        out_shape = jax.ShapeDtypeStruct((Nn // CW, M, CW), jnp.float32)
    else:
        out_spec = pl.BlockSpec((bm, bn), lambda m, n: (m, n))
        out_shape = jax.ShapeDtypeStruct((M, Nn), out_dtype)

    def body(*refs):
        it = iter(refs)
        lhs_ref = next(it)
        rhs_ref = next(it)
        bpre_ref = next(it) if bpre is not None else None
        bpost_ref = next(it) if bpost is not None else None
        out_ref = next(it)

        if chunked_lhs:
            a = jnp.concatenate([lhs_ref[c] for c in range(C)], axis=1)
            a = jnp.maximum(a + bpre_ref[...], 0.0)
        else:
            a = lhs_ref[...]
        a = a.astype(jnp.bfloat16)
        z = jnp.dot(a, rhs_ref[...], preferred_element_type=jnp.float32)
        if bpost is not None:
            z = z + bpost_ref[...]
        if act == "relu":
            z = jnp.maximum(z, 0.0)
        elif act == "leaky":
            z = jnp.where(z >= 0.0, z, 0.01 * z)
        if out_chunked:
            for co in range(CO):
                out_ref[co, :, :] = z[:, co * CW:(co + 1) * CW]
        else:
            out_ref[...] = z.astype(out_dtype)

    return pl.pallas_call(
        body,
        grid=(nm, nn),
        in_specs=in_specs,
        out_specs=out_spec,
        out_shape=out_shape,
        compiler_params=pltpu.CompilerParams(
            dimension_semantics=("parallel", "parallel")),
    )(*args)


def _segsum_sc(yt, srcs, dsts, nch):
    NB = srcs.shape[1]
    cpc = nch // NC
    rows = NP // NS

    mesh = plsc.VectorSubcoreMesh(core_axis_name="c", subcore_axis_name="s",
                                  num_cores=NC, num_subcores=NS)

    @functools.partial(
        pl.kernel,
        out_type=jax.ShapeDtypeStruct((nch * NP, CW), jnp.float32),
        mesh=mesh,
        scratch_types=[
            pltpu.VMEM_SHARED((NP, CW), jnp.float32),
            pltpu.VMEM((NB, EB), jnp.int32),
            pltpu.VMEM((NB, EB), jnp.int32),
            pltpu.VMEM((NB, EB), jnp.int32),
            pltpu.VMEM((2, EB, CW), jnp.float32),
            pltpu.SemaphoreType.DMA,
        ],
    )
    def seg_kernel(yt_h, src_h, dst_h, out_h, acc, srcv, dstv, sabs, buf, sem):
        c = lax.axis_index("c")
        s = lax.axis_index("s")
        pltpu.sync_copy(src_h.at[s], srcv)
        pltpu.sync_copy(dst_h.at[s], dstv)
        for cl in range(cpc):
            ci = c * cpc + cl
            base = ci * NP
            for j in range(NB):
                for q in range(EB // 16):
                    sabs[j, pl.ds(q * 16, 16)] = (
                        srcv[j, pl.ds(q * 16, 16)] + base)
            pltpu.sync_copy(yt_h.at[pl.ds(base + s * rows, rows)],
                            acc.at[pl.ds(s * rows, rows)])
            plsc.subcore_barrier()
            for b in range(NB):
                pltpu.async_copy(yt_h.at[sabs.at[b]], buf.at[b % 2],
                                 sem).wait()
                pltpu.sync_copy(buf.at[b % 2], acc.at[dstv.at[b]], add=True)
            plsc.subcore_barrier()
            pltpu.sync_copy(acc.at[pl.ds(s * rows, rows)],
                            out_h.at[pl.ds(base + s * rows, rows)])

    return seg_kernel(yt, srcs, dsts)


def _prep_edges(edge_index, n):
    src = edge_index[0].astype(jnp.int32)
    dst = edge_index[1].astype(jnp.int32)
    e = src.shape[0]
    ep = ((e + NS * EB - 1) // (NS * EB)) * (NS * EB)
    pad = ep - e
    pad_idx = n + (jnp.arange(pad, dtype=jnp.int32) % (NP - n))
    srcp = jnp.concatenate([src, pad_idx]).reshape(NS, ep // (NS * EB), EB)
    dstp = jnp.concatenate([dst, pad_idx]).reshape(NS, ep // (NS * EB), EB)
    return srcp, dstp


def kernel(x, edge_index, W1a, b1a, W1b, b1b, W2a, b2a, W2b, b2b,
           Wl, bl, Wd1, bd1, Wd2, bd2):
    n = x.shape[0]
    x_p = jnp.pad(x, ((0, NP - n), (0, 0))).astype(jnp.bfloat16)
    srcs, dsts = _prep_edges(edge_index, n)
    W1a, W1b, W2a, W2b, Wl, Wd1, Wd2 = (
        w.astype(jnp.bfloat16) for w in (W1a, W1b, W2a, W2b, Wl, Wd1, Wd2))

    y1 = _mm(x_p, W1a, out_chunked=True)
    h1pre = _segsum_sc(y1.reshape(-1, CW), srcs, dsts, W1a.shape[1] // CW)
    h1pre = h1pre.reshape(W1a.shape[1] // CW, NP, CW)

    (y2,) = _gin_mlp_mm(
        h1pre, [W1b, W2a],
        [b1a, b1b, jnp.zeros((W2a.shape[1],), jnp.float32)],
        ["relu", None], [None, True])
    h2pre = _segsum_sc(y2.reshape(-1, CW), srcs, dsts, W2a.shape[1] // CW)
    h2pre = h2pre.reshape(W2a.shape[1] // CW, NP, CW)

    enc, dec = _gin_mlp_mm(
        h2pre, [W2b, Wl, Wd1, Wd2],
        [b2a, b2b, bl, bd1, bd2],
        [None, None, "leaky", None], [None, False, None, False])

    return (dec[:n], enc[:n])


# speedup vs baseline: 2.7350x; 1.0610x over previous
import functools

import jax
import jax.numpy as jnp
from jax import lax
from jax.experimental import pallas as pl
from jax.experimental.pallas import tpu as pltpu
from jax.experimental.pallas import tpu_sc as plsc

NP = 10240
NC = 2
NS = 16
EB = 128
CW = 128


def _gin_mlp_mm(h_pre, Ws, bs, acts, out_chunked, out_dtype=jnp.float32,
                bm=512):
    C, M, _ = h_pre.shape
    K = C * CW
    nW = len(Ws)

    in_specs = [pl.BlockSpec((C, bm, CW), lambda m: (0, m, 0)),
                pl.BlockSpec((1, K), lambda m: (0, 0))]
    args = [h_pre, bs[0].reshape(1, K)]
    for W, b in zip(Ws, bs[1:]):
        kk, nn = W.shape
        in_specs.append(pl.BlockSpec((kk, nn), lambda m: (0, 0)))
        args.append(W)
        in_specs.append(pl.BlockSpec((1, nn), lambda m: (0, 0)))
        args.append(b.reshape(1, nn))

    out_specs = []
    out_shapes = []
    for i, oc in enumerate(out_chunked):
        nn = Ws[i].shape[1]
        if oc is None:
            continue
        if oc:
            out_specs.append(pl.BlockSpec((nn // CW, bm, CW),
                                          lambda m: (0, m, 0)))
            out_shapes.append(
                jax.ShapeDtypeStruct((nn // CW, M, CW), jnp.float32))
        else:
            out_specs.append(pl.BlockSpec((bm, nn), lambda m: (m, 0)))
            out_shapes.append(jax.ShapeDtypeStruct((M, nn), jnp.float32))

    def body(*refs):
        it = iter(refs)
        lhs_ref = next(it)
        bpre_ref = next(it)
        wrefs = []
        for _ in range(nW):
            wrefs.append((next(it), next(it)))
        orefs = [next(it) for _ in range(len(out_specs))]

        a = jnp.concatenate([lhs_ref[c] for c in range(C)], axis=1)
        a = jnp.maximum(a + bpre_ref[...], 0.0).astype(jnp.bfloat16)
        oi = 0
        for i, ((w_ref, b_ref), act) in enumerate(zip(wrefs, acts)):
            z = jnp.dot(a, w_ref[...], preferred_element_type=jnp.float32)
            z = z + b_ref[...]
            if act == "relu":
                z = jnp.maximum(z, 0.0)
            elif act == "leaky":
                z = jnp.where(z >= 0.0, z, 0.01 * z)
            oc = out_chunked[i]
            if oc is not None:
                if oc:
                    nn = z.shape[1]
                    for co in range(nn // CW):
                        orefs[oi][co, :, :] = z[:, co * CW:(co + 1) * CW]
                else:
                    orefs[oi][...] = z
                oi += 1
            a = z.astype(jnp.bfloat16)

    outs = pl.pallas_call(
        body,
        grid=(M // bm,),
        in_specs=in_specs,
        out_specs=out_specs,
        out_shape=out_shapes,
        compiler_params=pltpu.CompilerParams(
            dimension_semantics=("parallel",)),
    )(*args)
    return outs


def _mm(lhs, rhs, *, bpre=None, bpost=None, act=None, out_chunked=False,
        out_dtype=jnp.float32, bm=512, bn=2048):
    chunked_lhs = lhs.ndim == 3
    if chunked_lhs:
        K = lhs.shape[0] * CW
        M = lhs.shape[1]
    else:
        M, K = lhs.shape
    Nn = rhs.shape[1]
    bn = min(bn, Nn)
    nm, nn = M // bm, Nn // bn
    C = K // CW
    CO = bn // CW

    in_specs = []
    args = []
    if chunked_lhs:
        in_specs.append(pl.BlockSpec((C, bm, CW), lambda m, n: (0, m, 0)))
    else:
        in_specs.append(pl.BlockSpec((bm, K), lambda m, n: (m, 0)))
    args.append(lhs)
    in_specs.append(pl.BlockSpec((K, bn), lambda m, n: (0, n)))
    args.append(rhs)
    if bpre is not None:
        in_specs.append(pl.BlockSpec((1, K), lambda m, n: (0, 0)))
        args.append(bpre.reshape(1, K))
    if bpost is not None:
        in_specs.append(pl.BlockSpec((1, bn), lambda m, n: (0, n)))
        args.append(bpost.reshape(1, Nn))

    if out_chunked:
        out_spec = pl.BlockSpec((CO, bm, CW), lambda m, n: (n, m, 0))
        out_shape = jax.ShapeDtypeStruct((Nn // CW, M, CW), jnp.float32)
    else:
        out_spec = pl.BlockSpec((bm, bn), lambda m, n: (m, n))
        out_shape = jax.ShapeDtypeStruct((M, Nn), out_dtype)

    def body(*refs):
        it = iter(refs)
        lhs_ref = next(it)
        rhs_ref = next(it)
        bpre_ref = next(it) if bpre is not None else None
        bpost_ref = next(it) if bpost is not None else None
        out_ref = next(it)

        if chunked_lhs:
            a = jnp.concatenate([lhs_ref[c] for c in range(C)], axis=1)
            a = jnp.maximum(a + bpre_ref[...], 0.0)
        else:
            a = lhs_ref[...]
        a = a.astype(jnp.bfloat16)
        z = jnp.dot(a, rhs_ref[...], preferred_element_type=jnp.float32)
        if bpost is not None:
            z = z + bpost_ref[...]
        if act == "relu":
            z = jnp.maximum(z, 0.0)
        elif act == "leaky":
            z = jnp.where(z >= 0.0, z, 0.01 * z)
        if out_chunked:
            for co in range(CO):
                out_ref[co, :, :] = z[:, co * CW:(co + 1) * CW]
        else:
            out_ref[...] = z.astype(out_dtype)

    return pl.pallas_call(
        body,
        grid=(nm, nn),
        in_specs=in_specs,
        out_specs=out_spec,
        out_shape=out_shape,
        compiler_params=pltpu.CompilerParams(
            dimension_semantics=("parallel", "parallel")),
    )(*args)


def _segsum_sc(yt, srcs, dsts, nch):
    NB = srcs.shape[1]
    cpc = nch // NC
    rows = NP // NS

    mesh = plsc.VectorSubcoreMesh(core_axis_name="c", subcore_axis_name="s",
                                  num_cores=NC, num_subcores=NS)

    @functools.partial(
        pl.kernel,
        out_type=jax.ShapeDtypeStruct((nch * NP, CW), jnp.float32),
        mesh=mesh,
        scratch_types=[
            pltpu.VMEM_SHARED((NP, CW), jnp.float32),
            pltpu.VMEM((NB, EB), jnp.int32),
            pltpu.VMEM((NB, EB), jnp.int32),
            pltpu.VMEM((NB, EB), jnp.int32),
            pltpu.VMEM((2, EB, CW), jnp.float32),
            pltpu.SemaphoreType.DMA,
        ],
    )
    def seg_kernel(yt_h, src_h, dst_h, out_h, acc, srcv, dstv, sabs, buf, sem):
        c = lax.axis_index("c")
        s = lax.axis_index("s")
        pltpu.sync_copy(src_h.at[s], srcv)
        pltpu.sync_copy(dst_h.at[s], dstv)
        for cl in range(cpc):
            ci = c * cpc + cl
            base = ci * NP
            for j in range(NB):
                for q in range(EB // 16):
                    sabs[j, pl.ds(q * 16, 16)] = (
                        srcv[j, pl.ds(q * 16, 16)] + base)
            pltpu.sync_copy(yt_h.at[pl.ds(base + s * rows, rows)],
                            acc.at[pl.ds(s * rows, rows)])
            plsc.subcore_barrier()
            pltpu.async_copy(yt_h.at[sabs.at[0]], buf.at[0], sem).wait()
            for b in range(NB):
                if b + 1 < NB:
                    nxt = pltpu.async_copy(yt_h.at[sabs.at[b + 1]],
                                           buf.at[(b + 1) % 2], sem)
                pltpu.sync_copy(buf.at[b % 2], acc.at[dstv.at[b]], add=True)
                if b + 1 < NB:
                    nxt.wait()
            plsc.subcore_barrier()
            pltpu.sync_copy(acc.at[pl.ds(s * rows, rows)],
                            out_h.at[pl.ds(base + s * rows, rows)])

    return seg_kernel(yt, srcs, dsts)


def _prep_edges(edge_index, n):
    src = edge_index[0].astype(jnp.int32)
    dst = edge_index[1].astype(jnp.int32)
    e = src.shape[0]
    ep = ((e + NS * EB - 1) // (NS * EB)) * (NS * EB)
    pad = ep - e
    pad_idx = n + (jnp.arange(pad, dtype=jnp.int32) % (NP - n))
    srcp = jnp.concatenate([src, pad_idx]).reshape(NS, ep // (NS * EB), EB)
    dstp = jnp.concatenate([dst, pad_idx]).reshape(NS, ep // (NS * EB), EB)
    return srcp, dstp


def kernel(x, edge_index, W1a, b1a, W1b, b1b, W2a, b2a, W2b, b2b,
           Wl, bl, Wd1, bd1, Wd2, bd2):
    n = x.shape[0]
    x_p = jnp.pad(x, ((0, NP - n), (0, 0))).astype(jnp.bfloat16)
    srcs, dsts = _prep_edges(edge_index, n)
    W1a, W1b, W2a, W2b, Wl, Wd1, Wd2 = (
        w.astype(jnp.bfloat16) for w in (W1a, W1b, W2a, W2b, Wl, Wd1, Wd2))

    y1 = _mm(x_p, W1a, out_chunked=True)
    h1pre = _segsum_sc(y1.reshape(-1, CW), srcs, dsts, W1a.shape[1] // CW)
    h1pre = h1pre.reshape(W1a.shape[1] // CW, NP, CW)

    (y2,) = _gin_mlp_mm(
        h1pre, [W1b, W2a],
        [b1a, b1b, jnp.zeros((W2a.shape[1],), jnp.float32)],
        ["relu", None], [None, True])
    h2pre = _segsum_sc(y2.reshape(-1, CW), srcs, dsts, W2a.shape[1] // CW)
    h2pre = h2pre.reshape(W2a.shape[1] // CW, NP, CW)

    enc, dec = _gin_mlp_mm(
        h2pre, [W2b, Wl, Wd1, Wd2],
        [b2a, b2b, bl, bd1, bd2],
        [None, None, "leaky", None], [None, False, None, False])

    return (dec[:n], enc[:n])


# speedup vs baseline: 2.7401x; 1.0019x over previous
import functools

import jax
import jax.numpy as jnp
from jax import lax
from jax.experimental import pallas as pl
from jax.experimental.pallas import tpu as pltpu
from jax.experimental.pallas import tpu_sc as plsc

NP = 10240
NC = 2
NS = 16
EB = 128
CW = 128


def _gin_mlp_mm(h_pre, Ws, bs, acts, out_chunked, out_dtype=jnp.float32,
                bm=1024):
    C, M, _ = h_pre.shape
    K = C * CW
    nW = len(Ws)

    in_specs = [pl.BlockSpec((C, bm, CW), lambda m: (0, m, 0)),
                pl.BlockSpec((1, K), lambda m: (0, 0))]
    args = [h_pre, bs[0].reshape(1, K)]
    for W, b in zip(Ws, bs[1:]):
        kk, nn = W.shape
        in_specs.append(pl.BlockSpec((kk, nn), lambda m: (0, 0)))
        args.append(W)
        in_specs.append(pl.BlockSpec((1, nn), lambda m: (0, 0)))
        args.append(b.reshape(1, nn))

    out_specs = []
    out_shapes = []
    for i, oc in enumerate(out_chunked):
        nn = Ws[i].shape[1]
        if oc is None:
            continue
        if oc:
            out_specs.append(pl.BlockSpec((nn // CW, bm, CW),
                                          lambda m: (0, m, 0)))
            out_shapes.append(
                jax.ShapeDtypeStruct((nn // CW, M, CW), jnp.float32))
        else:
            out_specs.append(pl.BlockSpec((bm, nn), lambda m: (m, 0)))
            out_shapes.append(jax.ShapeDtypeStruct((M, nn), jnp.float32))

    def body(*refs):
        it = iter(refs)
        lhs_ref = next(it)
        bpre_ref = next(it)
        wrefs = []
        for _ in range(nW):
            wrefs.append((next(it), next(it)))
        orefs = [next(it) for _ in range(len(out_specs))]

        a = jnp.concatenate([lhs_ref[c] for c in range(C)], axis=1)
        a = jnp.maximum(a + bpre_ref[...], 0.0).astype(jnp.bfloat16)
        oi = 0
        for i, ((w_ref, b_ref), act) in enumerate(zip(wrefs, acts)):
            z = jnp.dot(a, w_ref[...], preferred_element_type=jnp.float32)
            z = z + b_ref[...]
            if act == "relu":
                z = jnp.maximum(z, 0.0)
            elif act == "leaky":
                z = jnp.where(z >= 0.0, z, 0.01 * z)
            oc = out_chunked[i]
            if oc is not None:
                if oc:
                    nn = z.shape[1]
                    for co in range(nn // CW):
                        orefs[oi][co, :, :] = z[:, co * CW:(co + 1) * CW]
                else:
                    orefs[oi][...] = z
                oi += 1
            a = z.astype(jnp.bfloat16)

    outs = pl.pallas_call(
        body,
        grid=(M // bm,),
        in_specs=in_specs,
        out_specs=out_specs,
        out_shape=out_shapes,
        compiler_params=pltpu.CompilerParams(
            dimension_semantics=("parallel",)),
    )(*args)
    return outs


def _mm(lhs, rhs, *, bpre=None, bpost=None, act=None, out_chunked=False,
        out_dtype=jnp.float32, bm=1024, bn=2048):
    chunked_lhs = lhs.ndim == 3
    if chunked_lhs:
        K = lhs.shape[0] * CW
        M = lhs.shape[1]
    else:
        M, K = lhs.shape
    Nn = rhs.shape[1]
    bn = min(bn, Nn)
    nm, nn = M // bm, Nn // bn
    C = K // CW
    CO = bn // CW

    in_specs = []
    args = []
    if chunked_lhs:
        in_specs.append(pl.BlockSpec((C, bm, CW), lambda m, n: (0, m, 0)))
    else:
        in_specs.append(pl.BlockSpec((bm, K), lambda m, n: (m, 0)))
    args.append(lhs)
    in_specs.append(pl.BlockSpec((K, bn), lambda m, n: (0, n)))
    args.append(rhs)
    if bpre is not None:
        in_specs.append(pl.BlockSpec((1, K), lambda m, n: (0, 0)))
        args.append(bpre.reshape(1, K))
    if bpost is not None:
        in_specs.append(pl.BlockSpec((1, bn), lambda m, n: (0, n)))
        args.append(bpost.reshape(1, Nn))

    if out_chunked:
        out_spec = pl.BlockSpec((CO, bm, CW), lambda m, n: (n, m, 0))
        out_shape = jax.ShapeDtypeStruct((Nn // CW, M, CW), jnp.float32)
    else:
        out_spec = pl.BlockSpec((bm, bn), lambda m, n: (m, n))
        out_shape = jax.ShapeDtypeStruct((M, Nn), out_dtype)

    def body(*refs):
        it = iter(refs)
        lhs_ref = next(it)
        rhs_ref = next(it)
        bpre_ref = next(it) if bpre is not None else None
        bpost_ref = next(it) if bpost is not None else None
        out_ref = next(it)

        if chunked_lhs:
            a = jnp.concatenate([lhs_ref[c] for c in range(C)], axis=1)
            a = jnp.maximum(a + bpre_ref[...], 0.0)
        else:
            a = lhs_ref[...]
        a = a.astype(jnp.bfloat16)
        z = jnp.dot(a, rhs_ref[...], preferred_element_type=jnp.float32)
        if bpost is not None:
            z = z + bpost_ref[...]
        if act == "relu":
            z = jnp.maximum(z, 0.0)
        elif act == "leaky":
            z = jnp.where(z >= 0.0, z, 0.01 * z)
        if out_chunked:
            for co in range(CO):
                out_ref[co, :, :] = z[:, co * CW:(co + 1) * CW]
        else:
            out_ref[...] = z.astype(out_dtype)

    return pl.pallas_call(
        body,
        grid=(nm, nn),
        in_specs=in_specs,
        out_specs=out_spec,
        out_shape=out_shape,
        compiler_params=pltpu.CompilerParams(
            dimension_semantics=("parallel", "parallel")),
    )(*args)


def _segsum_sc(yt, srcs, dsts, nch):
    NB = srcs.shape[1]
    cpc = nch // NC
    rows = NP // NS

    mesh = plsc.VectorSubcoreMesh(core_axis_name="c", subcore_axis_name="s",
                                  num_cores=NC, num_subcores=NS)

    @functools.partial(
        pl.kernel,
        out_type=jax.ShapeDtypeStruct((nch * NP, CW), jnp.float32),
        mesh=mesh,
        scratch_types=[
            pltpu.VMEM_SHARED((NP, CW), jnp.float32),
            pltpu.VMEM((NB, EB), jnp.int32),
            pltpu.VMEM((NB, EB), jnp.int32),
            pltpu.VMEM((NB, EB), jnp.int32),
            pltpu.VMEM((2, EB, CW), jnp.float32),
            pltpu.SemaphoreType.DMA,
        ],
    )
    def seg_kernel(yt_h, src_h, dst_h, out_h, acc, srcv, dstv, sabs, buf, sem):
        c = lax.axis_index("c")
        s = lax.axis_index("s")
        pltpu.sync_copy(src_h.at[s], srcv)
        pltpu.sync_copy(dst_h.at[s], dstv)
        for cl in range(cpc):
            ci = c * cpc + cl
            base = ci * NP
            for j in range(NB):
                for q in range(EB // 16):
                    sabs[j, pl.ds(q * 16, 16)] = (
                        srcv[j, pl.ds(q * 16, 16)] + base)
            pltpu.sync_copy(yt_h.at[pl.ds(base + s * rows, rows)],
                            acc.at[pl.ds(s * rows, rows)])
            plsc.subcore_barrier()
            pltpu.async_copy(yt_h.at[sabs.at[0]], buf.at[0], sem).wait()
            for b in range(NB):
                if b + 1 < NB:
                    nxt = pltpu.async_copy(yt_h.at[sabs.at[b + 1]],
                                           buf.at[(b + 1) % 2], sem)
                pltpu.sync_copy(buf.at[b % 2], acc.at[dstv.at[b]], add=True)
                if b + 1 < NB:
                    nxt.wait()
            plsc.subcore_barrier()
            pltpu.sync_copy(acc.at[pl.ds(s * rows, rows)],
                            out_h.at[pl.ds(base + s * rows, rows)])

    return seg_kernel(yt, srcs, dsts)


def _prep_edges(edge_index, n):
    src = edge_index[0].astype(jnp.int32)
    dst = edge_index[1].astype(jnp.int32)
    e = src.shape[0]
    ep = ((e + NS * EB - 1) // (NS * EB)) * (NS * EB)
    pad = ep - e
    pad_idx = n + (jnp.arange(pad, dtype=jnp.int32) % (NP - n))
    srcp = jnp.concatenate([src, pad_idx]).reshape(NS, ep // (NS * EB), EB)
    dstp = jnp.concatenate([dst, pad_idx]).reshape(NS, ep // (NS * EB), EB)
    return srcp, dstp


def kernel(x, edge_index, W1a, b1a, W1b, b1b, W2a, b2a, W2b, b2b,
           Wl, bl, Wd1, bd1, Wd2, bd2):
    n = x.shape[0]
    x_p = jnp.pad(x, ((0, NP - n), (0, 0))).astype(jnp.bfloat16)
    srcs, dsts = _prep_edges(edge_index, n)
    W1a, W1b, W2a, W2b, Wl, Wd1, Wd2 = (
        w.astype(jnp.bfloat16) for w in (W1a, W1b, W2a, W2b, Wl, Wd1, Wd2))

    y1 = _mm(x_p, W1a, out_chunked=True)
    h1pre = _segsum_sc(y1.reshape(-1, CW), srcs, dsts, W1a.shape[1] // CW)
    h1pre = h1pre.reshape(W1a.shape[1] // CW, NP, CW)

    (y2,) = _gin_mlp_mm(
        h1pre, [W1b, W2a],
        [b1a, b1b, jnp.zeros((W2a.shape[1],), jnp.float32)],
        ["relu", None], [None, True])
    h2pre = _segsum_sc(y2.reshape(-1, CW), srcs, dsts, W2a.shape[1] // CW)
    h2pre = h2pre.reshape(W2a.shape[1] // CW, NP, CW)

    enc, dec = _gin_mlp_mm(
        h2pre, [W2b, Wl, Wd1, Wd2],
        [b2a, b2b, bl, bd1, bd2],
        [None, None, "leaky", None], [None, False, None, False], bm=512)

    return (dec[:n], enc[:n])


# speedup vs baseline: 2.8799x; 1.0510x over previous
import functools

import jax
import jax.numpy as jnp
from jax import lax
from jax.experimental import pallas as pl
from jax.experimental.pallas import tpu as pltpu
from jax.experimental.pallas import tpu_sc as plsc

NP = 10240
NC = 2
NS = 16
EB = 128
CW = 128


def _gin_mlp_mm(h_pre, Ws, bs, acts, out_chunked, out_dtype=jnp.float32,
                bm=1024):
    if not isinstance(h_pre, (list, tuple)):
        h_pre = [h_pre]
    Cs = [hp.shape[0] for hp in h_pre]
    M = h_pre[0].shape[1]
    K = sum(Cs) * CW
    nW = len(Ws)

    in_specs = [pl.BlockSpec((Ci, bm, CW), lambda m: (0, m, 0)) for Ci in Cs]
    in_specs.append(pl.BlockSpec((1, K), lambda m: (0, 0)))
    args = list(h_pre) + [bs[0].reshape(1, K)]
    for W, b in zip(Ws, bs[1:]):
        kk, nn = W.shape
        in_specs.append(pl.BlockSpec((kk, nn), lambda m: (0, 0)))
        args.append(W)
        in_specs.append(pl.BlockSpec((1, nn), lambda m: (0, 0)))
        args.append(b.reshape(1, nn))

    out_specs = []
    out_shapes = []
    for i, oc in enumerate(out_chunked):
        nn = Ws[i].shape[1]
        if oc is None:
            continue
        if oc:
            out_specs.append(pl.BlockSpec((nn // CW, bm, CW),
                                          lambda m: (0, m, 0)))
            out_shapes.append(
                jax.ShapeDtypeStruct((nn // CW, M, CW), jnp.float32))
        else:
            out_specs.append(pl.BlockSpec((bm, nn), lambda m: (m, 0)))
            out_shapes.append(jax.ShapeDtypeStruct((M, nn), jnp.float32))

    def body(*refs):
        it = iter(refs)
        lhs_refs = [next(it) for _ in Cs]
        bpre_ref = next(it)
        wrefs = []
        for _ in range(nW):
            wrefs.append((next(it), next(it)))
        orefs = [next(it) for _ in range(len(out_specs))]

        a = jnp.concatenate(
            [lr[c] for lr, Ci in zip(lhs_refs, Cs) for c in range(Ci)],
            axis=1)
        a = jnp.maximum(a + bpre_ref[...], 0.0).astype(jnp.bfloat16)
        oi = 0
        for i, ((w_ref, b_ref), act) in enumerate(zip(wrefs, acts)):
            z = jnp.dot(a, w_ref[...], preferred_element_type=jnp.float32)
            z = z + b_ref[...]
            if act == "relu":
                z = jnp.maximum(z, 0.0)
            elif act == "leaky":
                z = jnp.where(z >= 0.0, z, 0.01 * z)
            oc = out_chunked[i]
            if oc is not None:
                if oc:
                    nn = z.shape[1]
                    for co in range(nn // CW):
                        orefs[oi][co, :, :] = z[:, co * CW:(co + 1) * CW]
                else:
                    orefs[oi][...] = z
                oi += 1
            a = z.astype(jnp.bfloat16)

    outs = pl.pallas_call(
        body,
        grid=(M // bm,),
        in_specs=in_specs,
        out_specs=out_specs,
        out_shape=out_shapes,
        compiler_params=pltpu.CompilerParams(
            dimension_semantics=("parallel",)),
    )(*args)
    return outs


def _mm(lhs, rhs, *, bpre=None, bpost=None, act=None, out_chunked=False,
        out_dtype=jnp.float32, bm=1024, bn=2048):
    chunked_lhs = lhs.ndim == 3
    if chunked_lhs:
        K = lhs.shape[0] * CW
        M = lhs.shape[1]
    else:
        M, K = lhs.shape
    Nn = rhs.shape[1]
    bn = min(bn, Nn)
    nm, nn = M // bm, Nn // bn
    C = K // CW
    CO = bn // CW

    in_specs = []
    args = []
    if chunked_lhs:
        in_specs.append(pl.BlockSpec((C, bm, CW), lambda m, n: (0, m, 0)))
    else:
        in_specs.append(pl.BlockSpec((bm, K), lambda m, n: (m, 0)))
    args.append(lhs)
    in_specs.append(pl.BlockSpec((K, bn), lambda m, n: (0, n)))
    args.append(rhs)
    if bpre is not None:
        in_specs.append(pl.BlockSpec((1, K), lambda m, n: (0, 0)))
        args.append(bpre.reshape(1, K))
    if bpost is not None:
        in_specs.append(pl.BlockSpec((1, bn), lambda m, n: (0, n)))
        args.append(bpost.reshape(1, Nn))

    if out_chunked:
        out_spec = pl.BlockSpec((CO, bm, CW), lambda m, n: (n, m, 0))
        out_shape = jax.ShapeDtypeStruct((Nn // CW, M, CW), jnp.float32)
    else:
        out_spec = pl.BlockSpec((bm, bn), lambda m, n: (m, n))
        out_shape = jax.ShapeDtypeStruct((M, Nn), out_dtype)

    def body(*refs):
        it = iter(refs)
        lhs_ref = next(it)
        rhs_ref = next(it)
        bpre_ref = next(it) if bpre is not None else None
        bpost_ref = next(it) if bpost is not None else None
        out_ref = next(it)

        if chunked_lhs:
            a = jnp.concatenate([lhs_ref[c] for c in range(C)], axis=1)
            a = jnp.maximum(a + bpre_ref[...], 0.0)
        else:
            a = lhs_ref[...]
        a = a.astype(jnp.bfloat16)
        z = jnp.dot(a, rhs_ref[...], preferred_element_type=jnp.float32)
        if bpost is not None:
            z = z + bpost_ref[...]
        if act == "relu":
            z = jnp.maximum(z, 0.0)
        elif act == "leaky":
            z = jnp.where(z >= 0.0, z, 0.01 * z)
        if out_chunked:
            for co in range(CO):
                out_ref[co, :, :] = z[:, co * CW:(co + 1) * CW]
        else:
            out_ref[...] = z.astype(out_dtype)

    return pl.pallas_call(
        body,
        grid=(nm, nn),
        in_specs=in_specs,
        out_specs=out_spec,
        out_shape=out_shape,
        compiler_params=pltpu.CompilerParams(
            dimension_semantics=("parallel", "parallel")),
    )(*args)


def _segsum_sc(yt, srcs, dsts, nch):
    NB = srcs.shape[1]
    cpc = nch // NC
    rows = NP // NS

    mesh = plsc.VectorSubcoreMesh(core_axis_name="c", subcore_axis_name="s",
                                  num_cores=NC, num_subcores=NS)

    @functools.partial(
        pl.kernel,
        out_type=jax.ShapeDtypeStruct((nch * NP, CW), jnp.float32),
        mesh=mesh,
        scratch_types=[
            pltpu.VMEM_SHARED((NP, CW), jnp.float32),
            pltpu.VMEM((NB, EB), jnp.int32),
            pltpu.VMEM((NB, EB), jnp.int32),
            pltpu.VMEM((NB, EB), jnp.int32),
            pltpu.VMEM((2, EB, CW), jnp.float32),
            pltpu.SemaphoreType.DMA,
        ],
    )
    def seg_kernel(yt_h, src_h, dst_h, out_h, acc, srcv, dstv, sabs, buf, sem):
        c = lax.axis_index("c")
        s = lax.axis_index("s")
        pltpu.sync_copy(src_h.at[s], srcv)
        pltpu.sync_copy(dst_h.at[s], dstv)
        for cl in range(cpc):
            ci = c * cpc + cl
            base = ci * NP
            for j in range(NB):
                for q in range(EB // 16):
                    sabs[j, pl.ds(q * 16, 16)] = (
                        srcv[j, pl.ds(q * 16, 16)] + base)
            pltpu.sync_copy(yt_h.at[pl.ds(base + s * rows, rows)],
                            acc.at[pl.ds(s * rows, rows)])
            plsc.subcore_barrier()
            pltpu.async_copy(yt_h.at[sabs.at[0]], buf.at[0], sem).wait()
            for b in range(NB):
                if b + 1 < NB:
                    nxt = pltpu.async_copy(yt_h.at[sabs.at[b + 1]],
                                           buf.at[(b + 1) % 2], sem)
                pltpu.sync_copy(buf.at[b % 2], acc.at[dstv.at[b]], add=True)
                if b + 1 < NB:
                    nxt.wait()
            plsc.subcore_barrier()
            pltpu.sync_copy(acc.at[pl.ds(s * rows, rows)],
                            out_h.at[pl.ds(base + s * rows, rows)])

    return seg_kernel(yt, srcs, dsts)


def _prep_edges(edge_index, n):
    src = edge_index[0].astype(jnp.int32)
    dst = edge_index[1].astype(jnp.int32)
    e = src.shape[0]
    ep = ((e + NS * EB - 1) // (NS * EB)) * (NS * EB)
    pad = ep - e
    pad_idx = n + (jnp.arange(pad, dtype=jnp.int32) % (NP - n))
    srcp = jnp.concatenate([src, pad_idx]).reshape(NS, ep // (NS * EB), EB)
    dstp = jnp.concatenate([dst, pad_idx]).reshape(NS, ep // (NS * EB), EB)
    return srcp, dstp


def kernel(x, edge_index, W1a, b1a, W1b, b1b, W2a, b2a, W2b, b2b,
           Wl, bl, Wd1, bd1, Wd2, bd2):
    n = x.shape[0]
    x_p = jnp.pad(x, ((0, NP - n), (0, 0))).astype(jnp.bfloat16)
    srcs, dsts = _prep_edges(edge_index, n)
    W1a, W1b, W2a, W2b, Wl, Wd1, Wd2 = (
        w.astype(jnp.bfloat16) for w in (W1a, W1b, W2a, W2b, Wl, Wd1, Wd2))

    half = W1a.shape[1] // 2
    y1a = _mm(x_p, W1a[:, :half], out_chunked=True)
    h1pre_a = _segsum_sc(y1a.reshape(-1, CW), srcs, dsts, half // CW)
    y1b = _mm(x_p, W1a[:, half:], out_chunked=True)
    h1pre_b = _segsum_sc(y1b.reshape(-1, CW), srcs, dsts, half // CW)
    h1pre_a = h1pre_a.reshape(half // CW, NP, CW)
    h1pre_b = h1pre_b.reshape(half // CW, NP, CW)

    (y2,) = _gin_mlp_mm(
        [h1pre_a, h1pre_b], [W1b, W2a],
        [b1a, b1b, jnp.zeros((W2a.shape[1],), jnp.float32)],
        ["relu", None], [None, True])
    h2pre = _segsum_sc(y2.reshape(-1, CW), srcs, dsts, W2a.shape[1] // CW)
    h2pre = h2pre.reshape(W2a.shape[1] // CW, NP, CW)

    enc, dec = _gin_mlp_mm(
        h2pre, [W2b, Wl, Wd1, Wd2],
        [b2a, b2b, bl, bd1, bd2],
        [None, None, "leaky", None], [None, False, None, False], bm=512)

    return (dec[:n], enc[:n])


# speedup vs baseline: 2.9092x; 1.0102x over previous
import functools

import jax
import jax.numpy as jnp
from jax import lax
from jax.experimental import pallas as pl
from jax.experimental.pallas import tpu as pltpu
from jax.experimental.pallas import tpu_sc as plsc

NP = 10240
NC = 2
NS = 16
EB = 128
CW = 128


def _gin_mlp_mm(h_pre, Ws, bs, acts, out_chunked, out_dtype=jnp.float32,
                bm=1024):
    if not isinstance(h_pre, (list, tuple)):
        h_pre = [h_pre]
    Cs = [hp.shape[0] for hp in h_pre]
    M = h_pre[0].shape[1]
    K = sum(Cs) * CW
    nW = len(Ws)

    in_specs = [pl.BlockSpec((Ci, bm, CW), lambda m: (0, m, 0)) for Ci in Cs]
    in_specs.append(pl.BlockSpec((1, K), lambda m: (0, 0)))
    args = list(h_pre) + [bs[0].reshape(1, K)]
    for W, b in zip(Ws, bs[1:]):
        kk, nn = W.shape
        in_specs.append(pl.BlockSpec((kk, nn), lambda m: (0, 0)))
        args.append(W)
        in_specs.append(pl.BlockSpec((1, nn), lambda m: (0, 0)))
        args.append(b.reshape(1, nn))

    out_specs = []
    out_shapes = []
    for i, oc in enumerate(out_chunked):
        nn = Ws[i].shape[1]
        if oc is None:
            continue
        if oc:
            out_specs.append(pl.BlockSpec((nn // CW, bm, CW),
                                          lambda m: (0, m, 0)))
            out_shapes.append(
                jax.ShapeDtypeStruct((nn // CW, M, CW), jnp.float32))
        else:
            out_specs.append(pl.BlockSpec((bm, nn), lambda m: (m, 0)))
            out_shapes.append(jax.ShapeDtypeStruct((M, nn), jnp.float32))

    def body(*refs):
        it = iter(refs)
        lhs_refs = [next(it) for _ in Cs]
        bpre_ref = next(it)
        wrefs = []
        for _ in range(nW):
            wrefs.append((next(it), next(it)))
        orefs = [next(it) for _ in range(len(out_specs))]

        a = jnp.concatenate(
            [lr[c] for lr, Ci in zip(lhs_refs, Cs) for c in range(Ci)],
            axis=1)
        a = jnp.maximum(a + bpre_ref[...], 0.0).astype(jnp.bfloat16)
        oi = 0
        for i, ((w_ref, b_ref), act) in enumerate(zip(wrefs, acts)):
            z = jnp.dot(a, w_ref[...], preferred_element_type=jnp.float32)
            z = z + b_ref[...]
            if act == "relu":
                z = jnp.maximum(z, 0.0)
            elif act == "leaky":
                z = jnp.where(z >= 0.0, z, 0.01 * z)
            oc = out_chunked[i]
            if oc is not None:
                if oc:
                    nn = z.shape[1]
                    for co in range(nn // CW):
                        orefs[oi][co, :, :] = z[:, co * CW:(co + 1) * CW]
                else:
                    orefs[oi][...] = z
                oi += 1
            a = z.astype(jnp.bfloat16)

    outs = pl.pallas_call(
        body,
        grid=(M // bm,),
        in_specs=in_specs,
        out_specs=out_specs,
        out_shape=out_shapes,
        compiler_params=pltpu.CompilerParams(
            dimension_semantics=("parallel",)),
    )(*args)
    return outs


def _mm(lhs, rhs, *, bpre=None, bpost=None, act=None, out_chunked=False,
        out_dtype=jnp.float32, bm=1024, bn=2048):
    chunked_lhs = lhs.ndim == 3
    if chunked_lhs:
        K = lhs.shape[0] * CW
        M = lhs.shape[1]
    else:
        M, K = lhs.shape
    Nn = rhs.shape[1]
    bn = min(bn, Nn)
    nm, nn = M // bm, Nn // bn
    C = K // CW
    CO = bn // CW

    in_specs = []
    args = []
    if chunked_lhs:
        in_specs.append(pl.BlockSpec((C, bm, CW), lambda m, n: (0, m, 0)))
    else:
        in_specs.append(pl.BlockSpec((bm, K), lambda m, n: (m, 0)))
    args.append(lhs)
    in_specs.append(pl.BlockSpec((K, bn), lambda m, n: (0, n)))
    args.append(rhs)
    if bpre is not None:
        in_specs.append(pl.BlockSpec((1, K), lambda m, n: (0, 0)))
        args.append(bpre.reshape(1, K))
    if bpost is not None:
        in_specs.append(pl.BlockSpec((1, bn), lambda m, n: (0, n)))
        args.append(bpost.reshape(1, Nn))

    if out_chunked:
        out_spec = pl.BlockSpec((CO, bm, CW), lambda m, n: (n, m, 0))
        out_shape = jax.ShapeDtypeStruct((Nn // CW, M, CW), jnp.float32)
    else:
        out_spec = pl.BlockSpec((bm, bn), lambda m, n: (m, n))
        out_shape = jax.ShapeDtypeStruct((M, Nn), out_dtype)

    def body(*refs):
        it = iter(refs)
        lhs_ref = next(it)
        rhs_ref = next(it)
        bpre_ref = next(it) if bpre is not None else None
        bpost_ref = next(it) if bpost is not None else None
        out_ref = next(it)

        if chunked_lhs:
            a = jnp.concatenate([lhs_ref[c] for c in range(C)], axis=1)
            a = jnp.maximum(a + bpre_ref[...], 0.0)
        else:
            a = lhs_ref[...]
        a = a.astype(jnp.bfloat16)
        z = jnp.dot(a, rhs_ref[...], preferred_element_type=jnp.float32)
        if bpost is not None:
            z = z + bpost_ref[...]
        if act == "relu":
            z = jnp.maximum(z, 0.0)
        elif act == "leaky":
            z = jnp.where(z >= 0.0, z, 0.01 * z)
        if out_chunked:
            for co in range(CO):
                out_ref[co, :, :] = z[:, co * CW:(co + 1) * CW]
        else:
            out_ref[...] = z.astype(out_dtype)

    return pl.pallas_call(
        body,
        grid=(nm, nn),
        in_specs=in_specs,
        out_specs=out_spec,
        out_shape=out_shape,
        compiler_params=pltpu.CompilerParams(
            dimension_semantics=("parallel", "parallel")),
    )(*args)


def _segsum_sc(yt, srcs, dsts, nch):
    NB = srcs.shape[1]
    cpc = nch // NC
    rows = NP // NS

    mesh = plsc.VectorSubcoreMesh(core_axis_name="c", subcore_axis_name="s",
                                  num_cores=NC, num_subcores=NS)

    @functools.partial(
        pl.kernel,
        out_type=jax.ShapeDtypeStruct((nch * NP, CW), jnp.float32),
        mesh=mesh,
        scratch_types=[
            pltpu.VMEM_SHARED((NP, CW), jnp.float32),
            pltpu.VMEM((NB, EB), jnp.int32),
            pltpu.VMEM((NB, EB), jnp.int32),
            pltpu.VMEM((NB, EB), jnp.int32),
            pltpu.VMEM((2, EB, CW), jnp.float32),
            pltpu.SemaphoreType.DMA,
        ],
    )
    def seg_kernel(yt_h, src_h, dst_h, out_h, acc, srcv, dstv, sabs, buf, sem):
        c = lax.axis_index("c")
        s = lax.axis_index("s")
        pltpu.sync_copy(src_h.at[s], srcv)
        pltpu.sync_copy(dst_h.at[s], dstv)
        for cl in range(cpc):
            ci = c * cpc + cl
            base = ci * NP
            for j in range(NB):
                for q in range(EB // 16):
                    sabs[j, pl.ds(q * 16, 16)] = (
                        srcv[j, pl.ds(q * 16, 16)] + base)
            pltpu.sync_copy(yt_h.at[pl.ds(base + s * rows, rows)],
                            acc.at[pl.ds(s * rows, rows)])
            plsc.subcore_barrier()
            pltpu.async_copy(yt_h.at[sabs.at[0]], buf.at[0], sem).wait()
            for b in range(NB):
                if b + 1 < NB:
                    nxt = pltpu.async_copy(yt_h.at[sabs.at[b + 1]],
                                           buf.at[(b + 1) % 2], sem)
                pltpu.sync_copy(buf.at[b % 2], acc.at[dstv.at[b]], add=True)
                if b + 1 < NB:
                    nxt.wait()
            plsc.subcore_barrier()
            pltpu.sync_copy(acc.at[pl.ds(s * rows, rows)],
                            out_h.at[pl.ds(base + s * rows, rows)])

    return seg_kernel(yt, srcs, dsts)


def _prep_edges(edge_index, n):
    src = edge_index[0].astype(jnp.int32)
    dst = edge_index[1].astype(jnp.int32)
    e = src.shape[0]
    ep = ((e + NS * EB - 1) // (NS * EB)) * (NS * EB)
    pad = ep - e
    pad_idx = n + (jnp.arange(pad, dtype=jnp.int32) % (NP - n))
    srcp = jnp.concatenate([src, pad_idx]).reshape(NS, ep // (NS * EB), EB)
    dstp = jnp.concatenate([dst, pad_idx]).reshape(NS, ep // (NS * EB), EB)
    return srcp, dstp


def kernel(x, edge_index, W1a, b1a, W1b, b1b, W2a, b2a, W2b, b2b,
           Wl, bl, Wd1, bd1, Wd2, bd2):
    n = x.shape[0]
    x_p = jnp.pad(x, ((0, NP - n), (0, 0))).astype(jnp.bfloat16)
    srcs, dsts = _prep_edges(edge_index, n)
    W1a, W1b, W2a, W2b, Wl, Wd1, Wd2 = (
        w.astype(jnp.bfloat16) for w in (W1a, W1b, W2a, W2b, Wl, Wd1, Wd2))

    qs = 4
    qw = W1a.shape[1] // qs
    h1pre_parts = []
    for q in range(qs):
        yq = _mm(x_p, W1a[:, q * qw:(q + 1) * qw], out_chunked=True)
        hq = _segsum_sc(yq.reshape(-1, CW), srcs, dsts, qw // CW)
        h1pre_parts.append(hq.reshape(qw // CW, NP, CW))

    (y2,) = _gin_mlp_mm(
        h1pre_parts, [W1b, W2a],
        [b1a, b1b, jnp.zeros((W2a.shape[1],), jnp.float32)],
        ["relu", None], [None, True])
    h2pre = _segsum_sc(y2.reshape(-1, CW), srcs, dsts, W2a.shape[1] // CW)
    h2pre = h2pre.reshape(W2a.shape[1] // CW, NP, CW)

    enc, dec = _gin_mlp_mm(
        h2pre, [W2b, Wl, Wd1, Wd2],
        [b2a, b2b, bl, bd1, bd2],
        [None, None, "leaky", None], [None, False, None, False], bm=512)

    return (dec[:n], enc[:n])


# speedup vs baseline: 3.0333x; 1.0426x over previous
import functools

import jax
import jax.numpy as jnp
from jax import lax
from jax.experimental import pallas as pl
from jax.experimental.pallas import tpu as pltpu
from jax.experimental.pallas import tpu_sc as plsc

NP = 10240
NC = 2
NS = 16
EB = 64
CW = 128


def _gin_mlp_mm(h_pre, Ws, bs, acts, out_chunked, out_dtype=jnp.float32,
                bm=1024):
    if not isinstance(h_pre, (list, tuple)):
        h_pre = [h_pre]
    Cs = [hp.shape[0] for hp in h_pre]
    M = h_pre[0].shape[1]
    K = sum(Cs) * CW
    nW = len(Ws)

    in_specs = [pl.BlockSpec((Ci, bm, CW), lambda m: (0, m, 0)) for Ci in Cs]
    in_specs.append(pl.BlockSpec((1, K), lambda m: (0, 0)))
    args = list(h_pre) + [bs[0].reshape(1, K)]
    for W, b in zip(Ws, bs[1:]):
        kk, nn = W.shape
        in_specs.append(pl.BlockSpec((kk, nn), lambda m: (0, 0)))
        args.append(W)
        in_specs.append(pl.BlockSpec((1, nn), lambda m: (0, 0)))
        args.append(b.reshape(1, nn))

    out_specs = []
    out_shapes = []
    for i, oc in enumerate(out_chunked):
        nn = Ws[i].shape[1]
        if oc is None:
            continue
        if oc:
            out_specs.append(pl.BlockSpec((nn // CW, bm, CW),
                                          lambda m: (0, m, 0)))
            out_shapes.append(
                jax.ShapeDtypeStruct((nn // CW, M, CW), jnp.float32))
        else:
            out_specs.append(pl.BlockSpec((bm, nn), lambda m: (m, 0)))
            out_shapes.append(jax.ShapeDtypeStruct((M, nn), jnp.float32))

    def body(*refs):
        it = iter(refs)
        lhs_refs = [next(it) for _ in Cs]
        bpre_ref = next(it)
        wrefs = []
        for _ in range(nW):
            wrefs.append((next(it), next(it)))
        orefs = [next(it) for _ in range(len(out_specs))]

        a = jnp.concatenate(
            [lr[c] for lr, Ci in zip(lhs_refs, Cs) for c in range(Ci)],
            axis=1)
        a = jnp.maximum(a + bpre_ref[...], 0.0).astype(jnp.bfloat16)
        oi = 0
        for i, ((w_ref, b_ref), act) in enumerate(zip(wrefs, acts)):
            z = jnp.dot(a, w_ref[...], preferred_element_type=jnp.float32)
            z = z + b_ref[...]
            if act == "relu":
                z = jnp.maximum(z, 0.0)
            elif act == "leaky":
                z = jnp.where(z >= 0.0, z, 0.01 * z)
            oc = out_chunked[i]
            if oc is not None:
                if oc:
                    nn = z.shape[1]
                    for co in range(nn // CW):
                        orefs[oi][co, :, :] = z[:, co * CW:(co + 1) * CW]
                else:
                    orefs[oi][...] = z
                oi += 1
            a = z.astype(jnp.bfloat16)

    outs = pl.pallas_call(
        body,
        grid=(M // bm,),
        in_specs=in_specs,
        out_specs=out_specs,
        out_shape=out_shapes,
        compiler_params=pltpu.CompilerParams(
            dimension_semantics=("parallel",)),
    )(*args)
    return outs


def _mm(lhs, rhs, *, bpre=None, bpost=None, act=None, out_chunked=False,
        out_dtype=jnp.float32, bm=1024, bn=2048):
    chunked_lhs = lhs.ndim == 3
    if chunked_lhs:
        K = lhs.shape[0] * CW
        M = lhs.shape[1]
    else:
        M, K = lhs.shape
    Nn = rhs.shape[1]
    bn = min(bn, Nn)
    nm, nn = M // bm, Nn // bn
    C = K // CW
    CO = bn // CW

    in_specs = []
    args = []
    if chunked_lhs:
        in_specs.append(pl.BlockSpec((C, bm, CW), lambda m, n: (0, m, 0)))
    else:
        in_specs.append(pl.BlockSpec((bm, K), lambda m, n: (m, 0)))
    args.append(lhs)
    in_specs.append(pl.BlockSpec((K, bn), lambda m, n: (0, n)))
    args.append(rhs)
    if bpre is not None:
        in_specs.append(pl.BlockSpec((1, K), lambda m, n: (0, 0)))
        args.append(bpre.reshape(1, K))
    if bpost is not None:
        in_specs.append(pl.BlockSpec((1, bn), lambda m, n: (0, n)))
        args.append(bpost.reshape(1, Nn))

    if out_chunked:
        out_spec = pl.BlockSpec((CO, bm, CW), lambda m, n: (n, m, 0))
        out_shape = jax.ShapeDtypeStruct((Nn // CW, M, CW), jnp.float32)
    else:
        out_spec = pl.BlockSpec((bm, bn), lambda m, n: (m, n))
        out_shape = jax.ShapeDtypeStruct((M, Nn), out_dtype)

    def body(*refs):
        it = iter(refs)
        lhs_ref = next(it)
        rhs_ref = next(it)
        bpre_ref = next(it) if bpre is not None else None
        bpost_ref = next(it) if bpost is not None else None
        out_ref = next(it)

        if chunked_lhs:
            a = jnp.concatenate([lhs_ref[c] for c in range(C)], axis=1)
            a = jnp.maximum(a + bpre_ref[...], 0.0)
        else:
            a = lhs_ref[...]
        a = a.astype(jnp.bfloat16)
        z = jnp.dot(a, rhs_ref[...], preferred_element_type=jnp.float32)
        if bpost is not None:
            z = z + bpost_ref[...]
        if act == "relu":
            z = jnp.maximum(z, 0.0)
        elif act == "leaky":
            z = jnp.where(z >= 0.0, z, 0.01 * z)
        if out_chunked:
            for co in range(CO):
                out_ref[co, :, :] = z[:, co * CW:(co + 1) * CW]
        else:
            out_ref[...] = z.astype(out_dtype)

    return pl.pallas_call(
        body,
        grid=(nm, nn),
        in_specs=in_specs,
        out_specs=out_spec,
        out_shape=out_shape,
        compiler_params=pltpu.CompilerParams(
            dimension_semantics=("parallel", "parallel")),
    )(*args)


def _segsum_sc(yt, srcs, dsts, nch):
    NB = srcs.shape[1]
    cpc = nch // NC
    rows = NP // NS

    mesh = plsc.VectorSubcoreMesh(core_axis_name="c", subcore_axis_name="s",
                                  num_cores=NC, num_subcores=NS)

    @functools.partial(
        pl.kernel,
        out_type=jax.ShapeDtypeStruct((nch * NP, CW), jnp.float32),
        mesh=mesh,
        scratch_types=[
            pltpu.VMEM_SHARED((NP, CW), jnp.float32),
            pltpu.VMEM((NB, EB), jnp.int32),
            pltpu.VMEM((NB, EB), jnp.int32),
            pltpu.VMEM((NB, EB), jnp.int32),
            pltpu.VMEM((4, EB, CW), jnp.float32),
            pltpu.SemaphoreType.DMA,
        ],
    )
    def seg_kernel(yt_h, src_h, dst_h, out_h, acc, srcv, dstv, sabs, buf, sem):
        c = lax.axis_index("c")
        s = lax.axis_index("s")
        pltpu.sync_copy(src_h.at[s], srcv)
        pltpu.sync_copy(dst_h.at[s], dstv)
        for cl in range(cpc):
            ci = c * cpc + cl
            base = ci * NP
            for j in range(NB):
                for q in range(EB // 16):
                    sabs[j, pl.ds(q * 16, 16)] = (
                        srcv[j, pl.ds(q * 16, 16)] + base)
            pltpu.sync_copy(yt_h.at[pl.ds(base + s * rows, rows)],
                            acc.at[pl.ds(s * rows, rows)])
            plsc.subcore_barrier()
            handles = {}
            for b in range(min(3, NB)):
                handles[b] = pltpu.async_copy(yt_h.at[sabs.at[b]],
                                              buf.at[b % 4], sem)
            for b in range(NB):
                handles[b].wait()
                if b + 3 < NB:
                    handles[b + 3] = pltpu.async_copy(
                        yt_h.at[sabs.at[b + 3]], buf.at[(b + 3) % 4], sem)
                pltpu.sync_copy(buf.at[b % 4], acc.at[dstv.at[b]], add=True)
            plsc.subcore_barrier()
            pltpu.sync_copy(acc.at[pl.ds(s * rows, rows)],
                            out_h.at[pl.ds(base + s * rows, rows)])

    return seg_kernel(yt, srcs, dsts)


def _prep_edges(edge_index, n):
    src = edge_index[0].astype(jnp.int32)
    dst = edge_index[1].astype(jnp.int32)
    e = src.shape[0]
    ep = ((e + NS * EB - 1) // (NS * EB)) * (NS * EB)
    pad = ep - e
    pad_idx = n + (jnp.arange(pad, dtype=jnp.int32) % (NP - n))
    srcp = jnp.concatenate([src, pad_idx]).reshape(NS, ep // (NS * EB), EB)
    dstp = jnp.concatenate([dst, pad_idx]).reshape(NS, ep // (NS * EB), EB)
    return srcp, dstp


def kernel(x, edge_index, W1a, b1a, W1b, b1b, W2a, b2a, W2b, b2b,
           Wl, bl, Wd1, bd1, Wd2, bd2):
    n = x.shape[0]
    x_p = jnp.pad(x, ((0, NP - n), (0, 0))).astype(jnp.bfloat16)
    srcs, dsts = _prep_edges(edge_index, n)
    W1a, W1b, W2a, W2b, Wl, Wd1, Wd2 = (
        w.astype(jnp.bfloat16) for w in (W1a, W1b, W2a, W2b, Wl, Wd1, Wd2))

    qs = 4
    qw = W1a.shape[1] // qs
    h1pre_parts = []
    for q in range(qs):
        yq = _mm(x_p, W1a[:, q * qw:(q + 1) * qw], out_chunked=True)
        hq = _segsum_sc(yq.reshape(-1, CW), srcs, dsts, qw // CW)
        h1pre_parts.append(hq.reshape(qw // CW, NP, CW))

    (y2,) = _gin_mlp_mm(
        h1pre_parts, [W1b, W2a],
        [b1a, b1b, jnp.zeros((W2a.shape[1],), jnp.float32)],
        ["relu", None], [None, True])
    h2pre = _segsum_sc(y2.reshape(-1, CW), srcs, dsts, W2a.shape[1] // CW)
    h2pre = h2pre.reshape(W2a.shape[1] // CW, NP, CW)

    enc, dec = _gin_mlp_mm(
        h2pre, [W2b, Wl, Wd1, Wd2],
        [b2a, b2b, bl, bd1, bd2],
        [None, None, "leaky", None], [None, False, None, False], bm=512)

    return (dec[:n], enc[:n])


# speedup vs baseline: 3.0512x; 1.0059x over previous
import functools

import jax
import jax.numpy as jnp
from jax import lax
from jax.experimental import pallas as pl
from jax.experimental.pallas import tpu as pltpu
from jax.experimental.pallas import tpu_sc as plsc

NP = 10240
NC = 2
NS = 16
EB = 64
CW = 128


def _gin_mlp_mm(h_pre, Ws, bs, acts, out_chunked, out_dtype=jnp.float32,
                bm=1024):
    if not isinstance(h_pre, (list, tuple)):
        h_pre = [h_pre]
    Cs = [hp.shape[0] for hp in h_pre]
    M = h_pre[0].shape[1]
    K = sum(Cs) * CW
    nW = len(Ws)

    in_specs = [pl.BlockSpec((Ci, bm, CW), lambda m: (0, m, 0)) for Ci in Cs]
    in_specs.append(pl.BlockSpec((1, K), lambda m: (0, 0)))
    args = list(h_pre) + [bs[0].reshape(1, K)]
    for W, b in zip(Ws, bs[1:]):
        kk, nn = W.shape
        in_specs.append(pl.BlockSpec((kk, nn), lambda m: (0, 0)))
        args.append(W)
        in_specs.append(pl.BlockSpec((1, nn), lambda m: (0, 0)))
        args.append(b.reshape(1, nn))

    out_specs = []
    out_shapes = []
    for i, oc in enumerate(out_chunked):
        nn = Ws[i].shape[1]
        if oc is None:
            continue
        if oc:
            out_specs.append(pl.BlockSpec((nn // CW, bm, CW),
                                          lambda m: (0, m, 0)))
            out_shapes.append(
                jax.ShapeDtypeStruct((nn // CW, M, CW), jnp.float32))
        else:
            out_specs.append(pl.BlockSpec((bm, nn), lambda m: (m, 0)))
            out_shapes.append(jax.ShapeDtypeStruct((M, nn), jnp.float32))

    def body(*refs):
        it = iter(refs)
        lhs_refs = [next(it) for _ in Cs]
        bpre_ref = next(it)
        wrefs = []
        for _ in range(nW):
            wrefs.append((next(it), next(it)))
        orefs = [next(it) for _ in range(len(out_specs))]

        a = jnp.concatenate(
            [lr[c] for lr, Ci in zip(lhs_refs, Cs) for c in range(Ci)],
            axis=1)
        a = jnp.maximum(a + bpre_ref[...], 0.0).astype(jnp.bfloat16)
        oi = 0
        for i, ((w_ref, b_ref), act) in enumerate(zip(wrefs, acts)):
            z = jnp.dot(a, w_ref[...], preferred_element_type=jnp.float32)
            z = z + b_ref[...]
            if act == "relu":
                z = jnp.maximum(z, 0.0)
            elif act == "leaky":
                z = jnp.where(z >= 0.0, z, 0.01 * z)
            oc = out_chunked[i]
            if oc is not None:
                if oc:
                    nn = z.shape[1]
                    for co in range(nn // CW):
                        orefs[oi][co, :, :] = z[:, co * CW:(co + 1) * CW]
                else:
                    orefs[oi][...] = z
                oi += 1
            a = z.astype(jnp.bfloat16)

    outs = pl.pallas_call(
        body,
        grid=(M // bm,),
        in_specs=in_specs,
        out_specs=out_specs,
        out_shape=out_shapes,
        compiler_params=pltpu.CompilerParams(
            dimension_semantics=("parallel",)),
    )(*args)
    return outs


def _mm(lhs, rhs, *, bpre=None, bpost=None, act=None, out_chunked=False,
        out_dtype=jnp.float32, bm=1024, bn=2048):
    chunked_lhs = lhs.ndim == 3
    if chunked_lhs:
        K = lhs.shape[0] * CW
        M = lhs.shape[1]
    else:
        M, K = lhs.shape
    Nn = rhs.shape[1]
    bn = min(bn, Nn)
    nm, nn = M // bm, Nn // bn
    C = K // CW
    CO = bn // CW

    in_specs = []
    args = []
    if chunked_lhs:
        in_specs.append(pl.BlockSpec((C, bm, CW), lambda m, n: (0, m, 0)))
    else:
        in_specs.append(pl.BlockSpec((bm, K), lambda m, n: (m, 0)))
    args.append(lhs)
    in_specs.append(pl.BlockSpec((K, bn), lambda m, n: (0, n)))
    args.append(rhs)
    if bpre is not None:
        in_specs.append(pl.BlockSpec((1, K), lambda m, n: (0, 0)))
        args.append(bpre.reshape(1, K))
    if bpost is not None:
        in_specs.append(pl.BlockSpec((1, bn), lambda m, n: (0, n)))
        args.append(bpost.reshape(1, Nn))

    if out_chunked:
        out_spec = pl.BlockSpec((CO, bm, CW), lambda m, n: (n, m, 0))
        out_shape = jax.ShapeDtypeStruct((Nn // CW, M, CW), jnp.float32)
    else:
        out_spec = pl.BlockSpec((bm, bn), lambda m, n: (m, n))
        out_shape = jax.ShapeDtypeStruct((M, Nn), out_dtype)

    def body(*refs):
        it = iter(refs)
        lhs_ref = next(it)
        rhs_ref = next(it)
        bpre_ref = next(it) if bpre is not None else None
        bpost_ref = next(it) if bpost is not None else None
        out_ref = next(it)

        if chunked_lhs:
            a = jnp.concatenate([lhs_ref[c] for c in range(C)], axis=1)
            a = jnp.maximum(a + bpre_ref[...], 0.0)
        else:
            a = lhs_ref[...]
        a = a.astype(jnp.bfloat16)
        z = jnp.dot(a, rhs_ref[...], preferred_element_type=jnp.float32)
        if bpost is not None:
            z = z + bpost_ref[...]
        if act == "relu":
            z = jnp.maximum(z, 0.0)
        elif act == "leaky":
            z = jnp.where(z >= 0.0, z, 0.01 * z)
        if out_chunked:
            for co in range(CO):
                out_ref[co, :, :] = z[:, co * CW:(co + 1) * CW]
        else:
            out_ref[...] = z.astype(out_dtype)

    return pl.pallas_call(
        body,
        grid=(nm, nn),
        in_specs=in_specs,
        out_specs=out_spec,
        out_shape=out_shape,
        compiler_params=pltpu.CompilerParams(
            dimension_semantics=("parallel", "parallel")),
    )(*args)


def _segsum_sc(yt, srcs, dsts, nch):
    NB = srcs.shape[1]
    cpc = nch // NC
    rows = NP // NS

    mesh = plsc.VectorSubcoreMesh(core_axis_name="c", subcore_axis_name="s",
                                  num_cores=NC, num_subcores=NS)

    @functools.partial(
        pl.kernel,
        out_type=jax.ShapeDtypeStruct((nch * NP, CW), jnp.float32),
        mesh=mesh,
        scratch_types=[
            pltpu.VMEM_SHARED((NP, CW), jnp.float32),
            pltpu.VMEM((NB, EB), jnp.int32),
            pltpu.VMEM((NB, EB), jnp.int32),
            pltpu.VMEM((NB, EB), jnp.int32),
            pltpu.VMEM((4, EB, CW), jnp.float32),
            pltpu.SemaphoreType.DMA,
            pltpu.SemaphoreType.DMA,
        ],
    )
    def seg_kernel(yt_h, src_h, dst_h, out_h, acc, srcv, dstv, sabs, buf,
                   sem, sem2):
        c = lax.axis_index("c")
        s = lax.axis_index("s")
        pltpu.sync_copy(src_h.at[s], srcv)
        pltpu.sync_copy(dst_h.at[s], dstv)
        for cl in range(cpc):
            ci = c * cpc + cl
            base = ci * NP
            for j in range(NB):
                for q in range(EB // 16):
                    sabs[j, pl.ds(q * 16, 16)] = (
                        srcv[j, pl.ds(q * 16, 16)] + base)
            init = pltpu.async_copy(yt_h.at[pl.ds(base + s * rows, rows)],
                                    acc.at[pl.ds(s * rows, rows)], sem2)
            handles = {}
            for b in range(min(3, NB)):
                handles[b] = pltpu.async_copy(yt_h.at[sabs.at[b]],
                                              buf.at[b % 4], sem)
            init.wait()
            plsc.subcore_barrier()
            for b in range(NB):
                handles[b].wait()
                if b + 3 < NB:
                    handles[b + 3] = pltpu.async_copy(
                        yt_h.at[sabs.at[b + 3]], buf.at[(b + 3) % 4], sem)
                pltpu.sync_copy(buf.at[b % 4], acc.at[dstv.at[b]], add=True)
            plsc.subcore_barrier()
            pltpu.sync_copy(acc.at[pl.ds(s * rows, rows)],
                            out_h.at[pl.ds(base + s * rows, rows)])

    return seg_kernel(yt, srcs, dsts)


def _prep_edges(edge_index, n):
    src = edge_index[0].astype(jnp.int32)
    dst = edge_index[1].astype(jnp.int32)
    e = src.shape[0]
    ep = ((e + NS * EB - 1) // (NS * EB)) * (NS * EB)
    pad = ep - e
    pad_idx = n + (jnp.arange(pad, dtype=jnp.int32) % (NP - n))
    srcp = jnp.concatenate([src, pad_idx]).reshape(NS, ep // (NS * EB), EB)
    dstp = jnp.concatenate([dst, pad_idx]).reshape(NS, ep // (NS * EB), EB)
    return srcp, dstp


def kernel(x, edge_index, W1a, b1a, W1b, b1b, W2a, b2a, W2b, b2b,
           Wl, bl, Wd1, bd1, Wd2, bd2):
    n = x.shape[0]
    x_p = jnp.pad(x, ((0, NP - n), (0, 0))).astype(jnp.bfloat16)
    srcs, dsts = _prep_edges(edge_index, n)
    W1a, W1b, W2a, W2b, Wl, Wd1, Wd2 = (
        w.astype(jnp.bfloat16) for w in (W1a, W1b, W2a, W2b, Wl, Wd1, Wd2))

    qs = 4
    qw = W1a.shape[1] // qs
    h1pre_parts = []
    for q in range(qs):
        yq = _mm(x_p, W1a[:, q * qw:(q + 1) * qw], out_chunked=True)
        hq = _segsum_sc(yq.reshape(-1, CW), srcs, dsts, qw // CW)
        h1pre_parts.append(hq.reshape(qw // CW, NP, CW))

    (y2,) = _gin_mlp_mm(
        h1pre_parts, [W1b, W2a],
        [b1a, b1b, jnp.zeros((W2a.shape[1],), jnp.float32)],
        ["relu", None], [None, True])
    h2pre = _segsum_sc(y2.reshape(-1, CW), srcs, dsts, W2a.shape[1] // CW)
    h2pre = h2pre.reshape(W2a.shape[1] // CW, NP, CW)

    enc, dec = _gin_mlp_mm(
        h2pre, [W2b, Wl, Wd1, Wd2],
        [b2a, b2b, bl, bd1, bd2],
        [None, None, "leaky", None], [None, False, None, False], bm=512)

    return (dec[:n], enc[:n])
